# Initial kernel scaffold; baseline (speedup 1.0000x reference)
import jax
import jax.numpy as jnp
from jax.experimental import pallas as pl


def kernel(x, edge_index):
    raise NotImplementedError("write your pallas kernel here")


# speedup vs baseline: 1.0168x; 1.0168x over previous
import functools

import jax
import jax.numpy as jnp
from jax import lax
from jax.experimental import pallas as pl
from jax.experimental.pallas import tpu as pltpu
from jax.experimental.pallas import tpu_sc as plsc

_N_NODES = 10000
_D = 128
_E = 320000
_NC = 2
_NS = 16
_NW = _NC * _NS
_EPW = _E // _NW
_C = 80
_NCHUNK = _EPW // _C
_G = _C // 16


def _edge_kernel(x_hbm, edge_hbm, out_hbm, sidx, didx, sbuf, dbuf, outv,
                 sem_s, sem_d):
    wid = lax.axis_index("s") * _NC + lax.axis_index("c")
    base = wid * _EPW
    pltpu.sync_copy(edge_hbm.at[pl.ds(base, _EPW)], sidx)
    pltpu.sync_copy(edge_hbm.at[pl.ds(_E + base, _EPW)], didx)

    rows0 = lax.broadcasted_iota(jnp.int32, (16,), 0)

    @pl.loop(0, _NCHUNK)
    def _chunk(c):
        off = c * _C
        cs = pltpu.async_copy(x_hbm.at[sidx.at[pl.ds(off, _C)]], sbuf, sem_s)
        cd = pltpu.async_copy(x_hbm.at[didx.at[pl.ds(off, _C)]], dbuf, sem_d)
        cs.wait()
        cd.wait()
        for g in range(_G):
            r = rows0 + (g * 16)
            acc = jnp.zeros((16,), jnp.float32)
            for d in range(_D):
                cv = jnp.full((16,), d, jnp.int32)
                s = plsc.load_gather(sbuf, [r, cv])
                t = plsc.load_gather(dbuf, [r, cv])
                acc = acc + s * t
            res = 1.0 / (1.0 + jnp.exp(-acc))
            outv[pl.ds(off + g * 16, 16)] = res

    pltpu.sync_copy(outv, out_hbm.at[pl.ds(base, _EPW)])


@jax.jit
def kernel(x, edge_index):
    mesh = plsc.VectorSubcoreMesh(core_axis_name="c", subcore_axis_name="s",
                                  num_cores=_NC, num_subcores=_NS)
    f = pl.kernel(
        _edge_kernel,
        out_type=jax.ShapeDtypeStruct((_E,), jnp.float32),
        mesh=mesh,
        compiler_params=pltpu.CompilerParams(needs_layout_passes=False),
        scratch_types=[
            pltpu.VMEM((_EPW,), jnp.int32),
            pltpu.VMEM((_EPW,), jnp.int32),
            pltpu.VMEM((_C, _D), jnp.float32),
            pltpu.VMEM((_C, _D), jnp.float32),
            pltpu.VMEM((_EPW,), jnp.float32),
            pltpu.SemaphoreType.DMA,
            pltpu.SemaphoreType.DMA,
        ],
    )
    return f(x, edge_index.reshape(2 * _E))


# speedup vs baseline: 1.1482x; 1.1292x over previous
import functools

import jax
import jax.numpy as jnp
from jax import lax
from jax.experimental import pallas as pl
from jax.experimental.pallas import tpu as pltpu
from jax.experimental.pallas import tpu_sc as plsc

_N_NODES = 10000
_D = 128
_E = 320000
_NC = 2
_NS = 16
_NW = _NC * _NS
_EPW = _E // _NW
_C = 16
_NCHUNK = _EPW // _C
_G = _C // 16


def _dot_chunk(sb, db, outv, off, rows0):
    for g in range(_G):
        r = rows0 + (g * 16)

        def qbody(q, carry, r=r, sb=sb, db=db):
            acc, col = carry
            for _ in range(32):
                s = plsc.load_gather(sb, [r, col])
                t = plsc.load_gather(db, [r, col])
                acc = acc + s * t
                col = col + 1
            return acc, col

        acc, _ = pl.loop(
            0, _D // 32,
            init_carry=(jnp.zeros((16,), jnp.float32),
                        jnp.zeros((16,), jnp.int32)),
        )(qbody)
        res = 1.0 / (1.0 + jnp.exp(-acc))
        outv[pl.ds(off + g * 16, 16)] = res


def _edge_kernel(x_hbm, edge_hbm, out_hbm, x_s, sidx, didx,
                 sb0, db0, sb1, db1, outv, ss0, sd0, ss1, sd1):
    cid = lax.axis_index("c")
    sid = lax.axis_index("s")
    wid = sid * _NC + cid
    base = wid * _EPW

    rows_per = 640
    @pl.when(sid < _NS - 1)
    def _():
        pltpu.sync_copy(x_hbm.at[pl.ds(sid * rows_per, rows_per)],
                        x_s.at[pl.ds(sid * rows_per, rows_per)])
    @pl.when(sid == _NS - 1)
    def _():
        last = (_NS - 1) * rows_per
        pltpu.sync_copy(x_hbm.at[pl.ds(last, _N_NODES - last)],
                        x_s.at[pl.ds(last, _N_NODES - last)])
    pltpu.sync_copy(edge_hbm.at[pl.ds(base, _EPW)], sidx)
    pltpu.sync_copy(edge_hbm.at[pl.ds(_E + base, _EPW)], didx)
    plsc.subcore_barrier()

    rows0 = lax.broadcasted_iota(jnp.int32, (16,), 0)

    def start(cc, sb, db, ss, sd):
        o = cc * _C
        pltpu.async_copy(x_s.at[sidx.at[pl.ds(o, _C)]], sb, ss)
        pltpu.async_copy(x_s.at[didx.at[pl.ds(o, _C)]], db, sd)

    def wait(sb, db, ss, sd):
        pltpu.make_async_copy(x_s.at[pl.ds(0, _C)], sb, ss).wait()
        pltpu.make_async_copy(x_s.at[pl.ds(0, _C)], db, sd).wait()

    slots = ((sb0, db0, ss0, sd0), (sb1, db1, ss1, sd1))
    start(0, *slots[0])
    start(1, *slots[1])

    @pl.loop(0, _NCHUNK - 1, step=2)
    def _pair(c):
        for par in range(2):
            sb, db, ss, sd = slots[par]
            cc = c + par
            wait(sb, db, ss, sd)
            _dot_chunk(sb, db, outv, cc * _C, rows0)

            @pl.when(cc + 2 < _NCHUNK)
            def _(cc=cc, sb=sb, db=db, ss=ss, sd=sd):
                start(cc + 2, sb, db, ss, sd)

    wait(*slots[0])
    _dot_chunk(sb0, db0, outv, (_NCHUNK - 1) * _C, rows0)

    pltpu.sync_copy(outv, out_hbm.at[pl.ds(base, _EPW)])


@jax.jit
def kernel(x, edge_index):
    mesh = plsc.VectorSubcoreMesh(core_axis_name="c", subcore_axis_name="s",
                                  num_cores=_NC, num_subcores=_NS)
    f = pl.kernel(
        _edge_kernel,
        out_type=jax.ShapeDtypeStruct((_E,), jnp.float32),
        mesh=mesh,
        compiler_params=pltpu.CompilerParams(needs_layout_passes=False),
        scratch_types=[
            pltpu.VMEM_SHARED((_N_NODES, _D), jnp.float32),
            pltpu.VMEM((_EPW,), jnp.int32),
            pltpu.VMEM((_EPW,), jnp.int32),
            pltpu.VMEM((_C, _D), jnp.float32),
            pltpu.VMEM((_C, _D), jnp.float32),
            pltpu.VMEM((_C, _D), jnp.float32),
            pltpu.VMEM((_C, _D), jnp.float32),
            pltpu.VMEM((_EPW,), jnp.float32),
            pltpu.SemaphoreType.DMA,
            pltpu.SemaphoreType.DMA,
            pltpu.SemaphoreType.DMA,
            pltpu.SemaphoreType.DMA,
        ],
    )
    return f(x, edge_index.reshape(2 * _E))


# speedup vs baseline: 5.7963x; 5.0483x over previous
import functools

import jax
import jax.numpy as jnp
from jax import lax
from jax.experimental import pallas as pl
from jax.experimental.pallas import tpu as pltpu
from jax.experimental.pallas import tpu_sc as plsc

_N_NODES = 10000
_D = 128
_E = 320000
_NC = 2
_NS = 16
_NW = _NC * _NS
_EPW = _E // _NW
_C = 16
_NCHUNK = _EPW // _C
_G = _C // 16


def _dot_chunk(sb, db, outv, dots, off, last_lane):
    for e in range(_C):
        p = sb[e, pl.ds(0, 16)] * db[e, pl.ds(0, 16)]
        for c in range(1, _D // 16):
            p = p + sb[e, pl.ds(c * 16, 16)] * db[e, pl.ds(c * 16, 16)]
        cum = plsc.cumsum(p)
        plsc.store_scatter(dots, [jnp.full((16,), e, jnp.int32)], cum,
                           mask=last_lane)
    v = dots[...]
    outv[pl.ds(off, 16)] = 1.0 / (1.0 + jnp.exp(-v))


def _edge_kernel(x_hbm, edge_hbm, out_hbm, x_s, sidx, didx,
                 sb0, db0, sb1, db1, outv, dots, ss0, sd0, ss1, sd1):
    cid = lax.axis_index("c")
    sid = lax.axis_index("s")
    wid = sid * _NC + cid
    base = wid * _EPW

    rows_per = 640
    @pl.when(sid < _NS - 1)
    def _():
        pltpu.sync_copy(x_hbm.at[pl.ds(sid * rows_per, rows_per)],
                        x_s.at[pl.ds(sid * rows_per, rows_per)])
    @pl.when(sid == _NS - 1)
    def _():
        last = (_NS - 1) * rows_per
        pltpu.sync_copy(x_hbm.at[pl.ds(last, _N_NODES - last)],
                        x_s.at[pl.ds(last, _N_NODES - last)])
    pltpu.sync_copy(edge_hbm.at[pl.ds(base, _EPW)], sidx)
    pltpu.sync_copy(edge_hbm.at[pl.ds(_E + base, _EPW)], didx)
    plsc.subcore_barrier()

    last_lane = lax.broadcasted_iota(jnp.int32, (16,), 0) == 15

    def start(cc, sb, db, ss, sd):
        o = cc * _C
        pltpu.async_copy(x_s.at[sidx.at[pl.ds(o, _C)]], sb, ss)
        pltpu.async_copy(x_s.at[didx.at[pl.ds(o, _C)]], db, sd)

    def wait(sb, db, ss, sd):
        pltpu.make_async_copy(x_s.at[pl.ds(0, _C)], sb, ss).wait()
        pltpu.make_async_copy(x_s.at[pl.ds(0, _C)], db, sd).wait()

    slots = ((sb0, db0, ss0, sd0), (sb1, db1, ss1, sd1))
    start(0, *slots[0])
    start(1, *slots[1])

    @pl.loop(0, _NCHUNK - 1, step=2)
    def _pair(c):
        for par in range(2):
            sb, db, ss, sd = slots[par]
            cc = c + par
            wait(sb, db, ss, sd)
            _dot_chunk(sb, db, outv, dots, cc * _C, last_lane)

            @pl.when(cc + 2 < _NCHUNK)
            def _(cc=cc, sb=sb, db=db, ss=ss, sd=sd):
                start(cc + 2, sb, db, ss, sd)

    wait(*slots[0])
    _dot_chunk(sb0, db0, outv, dots, (_NCHUNK - 1) * _C, last_lane)

    pltpu.sync_copy(outv, out_hbm.at[pl.ds(base, _EPW)])


@jax.jit
def kernel(x, edge_index):
    mesh = plsc.VectorSubcoreMesh(core_axis_name="c", subcore_axis_name="s",
                                  num_cores=_NC, num_subcores=_NS)
    f = pl.kernel(
        _edge_kernel,
        out_type=jax.ShapeDtypeStruct((_E,), jnp.float32),
        mesh=mesh,
        compiler_params=pltpu.CompilerParams(needs_layout_passes=False),
        scratch_types=[
            pltpu.VMEM_SHARED((_N_NODES, _D), jnp.float32),
            pltpu.VMEM((_EPW,), jnp.int32),
            pltpu.VMEM((_EPW,), jnp.int32),
            pltpu.VMEM((_C, _D), jnp.float32),
            pltpu.VMEM((_C, _D), jnp.float32),
            pltpu.VMEM((_C, _D), jnp.float32),
            pltpu.VMEM((_C, _D), jnp.float32),
            pltpu.VMEM((_EPW,), jnp.float32),
            pltpu.VMEM((16,), jnp.float32),
            pltpu.SemaphoreType.DMA,
            pltpu.SemaphoreType.DMA,
            pltpu.SemaphoreType.DMA,
            pltpu.SemaphoreType.DMA,
        ],
    )
    return f(x, edge_index.reshape(2 * _E))
